# baseline (speedup 1.0000x reference)
import functools
import math

import jax
import jax.numpy as jnp
from jax import lax
from jax.experimental import pallas as pl
from jax.experimental.pallas import tpu as pltpu
from jax.experimental.pallas import tpu_sc as plsc


def _make_kernel(V2, D, S, B, NC, NS, L):
    NW = NC * NS
    T = S * B
    t_per_w = T // NW
    n_groups = t_per_w // L
    scale = math.sqrt(D)
    mesh = plsc.VectorSubcoreMesh(core_axis_name="c", subcore_axis_name="s")

    @functools.partial(
        pl.kernel,
        mesh=mesh,
        compiler_params=pltpu.CompilerParams(use_tc_tiling_on_sc=True),
        out_type=jax.ShapeDtypeStruct((T // 2, 2 * D), jnp.float32),
        scratch_types=[
            pltpu.VMEM((t_per_w,), jnp.int32),
            pltpu.VMEM((t_per_w, 2 * D), jnp.float32),
            pltpu.VMEM((t_per_w // 2, 2 * D), jnp.float32),
            pltpu.SemaphoreType.DMA,
        ],
    )
    def emb_kernel(idx_hbm, table2_hbm, out_hbm, idx_v, pairbuf, tokbuf, sem):
        wid = lax.axis_index("c") * NS + lax.axis_index("s")
        base = pl.multiple_of(wid * t_per_w, t_per_w)
        obase = pl.multiple_of(wid * (t_per_w // 2), t_per_w // 2)

        pltpu.sync_copy(idx_hbm.at[pl.ds(base, t_per_w)], idx_v)

        def fetch(g, _):
            t0 = g * L
            vb = lax.shift_right_logical(idx_v[pl.ds(t0, L)], 1)
            pltpu.async_copy(
                table2_hbm.at[vb], pairbuf.at[pl.ds(t0, L)], sem
            )
            return ()

        lax.fori_loop(0, n_groups, fetch, ())
        pltpu.make_async_copy(
            table2_hbm.at[pl.ds(0, t_per_w)], pairbuf, sem
        ).wait()

        def extract(g, _):
            t0 = g * L
            parity = idx_v[pl.ds(t0, L)] & 1
            for k in range(L):
                t = t0 + k
                odd = parity[k] == 1
                row = t0 // 2 + k // 2
                c0 = (k % 2) * D
                for j in range(D // L):
                    lo = pairbuf[t, pl.ds(j * L, L)]
                    hi = pairbuf[t, pl.ds(D + j * L, L)]
                    tokbuf[row, pl.ds(c0 + j * L, L)] = (
                        jnp.where(odd, hi, lo) * scale
                    )
            return ()

        lax.fori_loop(0, n_groups, extract, ())

        pltpu.sync_copy(tokbuf, out_hbm.at[pl.ds(obase, t_per_w // 2)])

    return emb_kernel


def kernel(tokens, embedding):
    S, B = tokens.shape
    V, D = embedding.shape
    info = plsc.get_sparse_core_info()
    NC, NS, L = info.num_cores, info.num_subcores, info.num_lanes
    idx = tokens.T.reshape(S * B).astype(jnp.int32)
    table2 = jnp.concatenate(
        [embedding[0::2, :], embedding[1::2, :]], axis=1
    )
    emb_kernel = _make_kernel(V // 2, D, S, B, NC, NS, L)
    out = emb_kernel(idx, table2)
    return out.reshape(B, S, D).transpose(1, 0, 2)


# speedup vs baseline: 22.2096x; 22.2096x over previous
import functools
import math

import jax
import jax.numpy as jnp
from jax import lax
from jax.experimental import pallas as pl
from jax.experimental.pallas import tpu as pltpu
from jax.experimental.pallas import tpu_sc as plsc


def _make_kernel(V, D, S, B, NC, NS, L):
    NW = NC * NS
    T = S * B
    t_per_w = T // NW
    C = 32
    n_chunks = t_per_w // C
    scale = math.sqrt(D)
    mesh = plsc.VectorSubcoreMesh(core_axis_name="c", subcore_axis_name="s")

    @functools.partial(
        pl.kernel,
        mesh=mesh,
        compiler_params=pltpu.CompilerParams(use_tc_tiling_on_sc=True),
        out_type=jax.ShapeDtypeStruct((T // 2, 2 * D), jnp.float32),
        scratch_types=[
            pltpu.VMEM((t_per_w,), jnp.int32),
            pltpu.VMEM((C * 8, D), jnp.float32),
            pltpu.VMEM((C * 8, D), jnp.float32),
            pltpu.VMEM((t_per_w // 2, 2 * D), jnp.float32),
            pltpu.SemaphoreType.DMA,
            pltpu.SemaphoreType.DMA,
        ],
    )
    def emb_kernel(
        idx_hbm, table_hbm, out_hbm, idx_v, stage_a, stage_b, tokbuf,
        sem_a, sem_b,
    ):
        wid = lax.axis_index("c") * NS + lax.axis_index("s")
        base = pl.multiple_of(wid * t_per_w, t_per_w)
        obase = pl.multiple_of(wid * (t_per_w // 2), t_per_w // 2)

        pltpu.sync_copy(idx_hbm.at[pl.ds(base, t_per_w)], idx_v)

        def fire(c, stage, sem):
            c0 = c * C

            def body(g, _):
                vec = idx_v[pl.ds(c0 + g * L, L)]
                for k in range(L):
                    v = vec[k]
                    g8 = pl.multiple_of(
                        lax.shift_right_logical(v, 3) * 8, 8
                    )
                    pltpu.async_copy(
                        table_hbm.at[pl.ds(g8, 8), :],
                        stage.at[pl.ds((g * L + k) * 8, 8), :],
                        sem,
                    )
                return ()

            lax.fori_loop(0, C // L, body, ())

        def drain_extract(c, stage, sem):
            pltpu.make_async_copy(
                table_hbm.at[pl.ds(0, C * 8), :], stage, sem
            ).wait()
            c0 = c * C

            def body(g, _):
                t0 = c0 + g * L
                vec = idx_v[pl.ds(t0, L)]
                for k in range(L):
                    v = vec[k]
                    r = (g * L + k) * 8 + (v & 7)
                    row = t0 // 2 + k // 2
                    col = (k % 2) * D
                    for j in range(D // L):
                        tokbuf[row, pl.ds(col + j * L, L)] = (
                            stage[r, pl.ds(j * L, L)] * scale
                        )
                return ()

            lax.fori_loop(0, C // L, body, ())

        fire(0, stage_a, sem_a)

        def pair_body(p, _):
            c = p * 2
            fire(c + 1, stage_b, sem_b)
            drain_extract(c, stage_a, sem_a)

            @pl.when(p < n_chunks // 2 - 1)
            def _():
                fire(c + 2, stage_a, sem_a)

            drain_extract(c + 1, stage_b, sem_b)
            return ()

        lax.fori_loop(0, n_chunks // 2, pair_body, ())

        pltpu.sync_copy(tokbuf, out_hbm.at[pl.ds(obase, t_per_w // 2)])

    return emb_kernel


def kernel(tokens, embedding):
    S, B = tokens.shape
    V, D = embedding.shape
    info = plsc.get_sparse_core_info()
    NC, NS, L = info.num_cores, info.num_subcores, info.num_lanes
    idx = tokens.T.reshape(S * B).astype(jnp.int32)
    emb_kernel = _make_kernel(V, D, S, B, NC, NS, L)
    out = emb_kernel(idx, embedding)
    return out.reshape(B, S, D).transpose(1, 0, 2)
